# baseline (speedup 1.0000x reference)
import jax
import jax.numpy as jnp
from jax import lax
from jax.experimental import pallas as pl
from jax.experimental.pallas import tpu as pltpu
from jax.experimental.pallas import tpu_sc as plsc

_VOCAB = 1000000
_MAXLEN = 200
_DIM = 64
_BATCH = 4096

_NC, _NS = 2, 16
_NW = _NC * _NS
_BPW = _BATCH // _NW

_TBLK = 1024
_TGRID = (_VOCAB + _TBLK - 1) // _TBLK
_TROWS = _TGRID * 8 * 128


def _tbody(in_ref, out_ref):
    t3 = in_ref[...].T.reshape(8, 128, 64)
    out_ref[:, :, 0:64] = t3[:, 0:64, :]
    out_ref[:, :, 64:128] = t3[:, 64:128, :]


def _tc_transpose(tt_cm):
    return pl.pallas_call(
        _tbody,
        out_shape=jax.ShapeDtypeStruct((_TGRID * 8, 64, 128), jnp.float32),
        grid=(_TGRID,),
        in_specs=[pl.BlockSpec((64, _TBLK), lambda g: (0, g))],
        out_specs=pl.BlockSpec((8, 64, 128), lambda g: (g, 0, 0)),
    )(tt_cm)


def _body(xt_hbm, tok_hbm, pos_hbm, q_hbm,
          xv, pos_v, r0, r1, o0, o1, g0, g1, w0, w1):
    wid = lax.axis_index("s") * _NC + lax.axis_index("c")
    b0 = wid * _BPW
    rows = (r0, r1)
    outs = (o0, o1)
    gsems = (g0, g1)
    wsems = (w0, w1)

    pltpu.sync_copy(xt_hbm.at[:, pl.ds(b0, _BPW)], xv)
    pltpu.sync_copy(pos_hbm, pos_v)

    def remap_row(s2, carry):
        for kk in range(_BPW // 16):
            d = pl.ds(kk * 16, 16)
            v = xv[s2, d]
            xv[s2, d] = (((v >> 7) << 7) | ((v & 63) << 1) | ((v >> 6) & 1))
        return carry

    lax.fori_loop(0, _MAXLEN, remap_row, 0)

    def issue_gather(s, b):
        pltpu.async_copy(tok_hbm.at[xv.at[s]], rows[b], gsems[b])

    def wait_gather(s, b):
        pltpu.make_async_copy(tok_hbm.at[xv.at[s]], rows[b], gsems[b]).wait()

    def issue_write(s, b):
        pltpu.async_copy(outs[b], q_hbm.at[s, :, wid, :, :], wsems[b])

    def wait_write(s, b):
        pltpu.make_async_copy(outs[b], q_hbm.at[s, :, wid, :, :],
                              wsems[b]).wait()

    lane = lax.iota(jnp.int32, 16)
    issue_gather(0, 0)

    def s_body(s, carry):
        for b in range(2):
            sb = s * 2 + b
            pl.when(sb + 1 < _MAXLEN)(lambda: issue_gather(sb + 1, 1 - b))
            wait_gather(sb, b)
            pl.when(sb >= 2)(lambda: wait_write(sb - 2, b))

            pvs = [pos_v[sb, pl.ds(k * 16, 16)] for k in range(_DIM // 16)]
            cols = [16 * k + lane for k in range(_DIM // 16)]
            i0s = [(16 * k + lane) >> 3 for k in range(_DIM // 16)]
            i1 = lane & 7

            def j_body(j, c2):
                rowmix = (lane + j) & 15
                trows = [rowmix + 16 * g for g in range(_BPW // 16)]
                for ks in ((0, 1), (2, 3)):
                    vals = []
                    for g in range(_BPW // 16):
                        for k in ks:
                            vals.append(
                                (g, k,
                                 plsc.load_gather(rows[b],
                                                  [trows[g], cols[k]])
                                 + pvs[k]))
                    for g, k, v in vals:
                        plsc.store_scatter(outs[b], [i0s[k], i1, trows[g]], v)
                return c2

            lax.fori_loop(0, 16, j_body, 0)
            issue_write(sb, b)
        return carry

    lax.fori_loop(0, _MAXLEN // 2, s_body, 0)
    wait_write(_MAXLEN - 2, 0)
    wait_write(_MAXLEN - 1, 1)


@jax.jit
def _embed(xt, tt_sc, pos_table):
    mesh = plsc.VectorSubcoreMesh(core_axis_name="c", subcore_axis_name="s")
    run = pl.kernel(
        _body,
        out_type=jax.ShapeDtypeStruct(
            (_MAXLEN, _DIM // 8, _NW, 8, _BPW), jnp.float32),
        mesh=mesh,
        scratch_types=(
            [pltpu.VMEM((_MAXLEN, _BPW), jnp.int32),
             pltpu.VMEM((_MAXLEN, _DIM), jnp.float32)]
            + [pltpu.VMEM((_BPW, _DIM), jnp.float32)] * 2
            + [pltpu.VMEM((_DIM // 8, 8, _BPW), jnp.float32)] * 2
            + [pltpu.SemaphoreType.DMA] * 4
        ),
        compiler_params=pltpu.CompilerParams(
            use_tc_tiling_on_sc=False, needs_layout_passes=False),
    )
    return run(xt, tt_sc, pos_table)


def kernel(x, token_table, pos_table):
    xt = x.T.astype(jnp.int32)
    tt3 = _tc_transpose(token_table.T)
    tt_sc = tt3.reshape(_TROWS, _DIM)
    q = _embed(xt, tt_sc, pos_table)
    return q.transpose(2, 4, 0, 1, 3).reshape(_BATCH, _MAXLEN, _DIM)


# speedup vs baseline: 1.3423x; 1.3423x over previous
import jax
import jax.numpy as jnp
from jax import lax
from jax.experimental import pallas as pl
from jax.experimental.pallas import tpu as pltpu
from jax.experimental.pallas import tpu_sc as plsc

_VOCAB = 1000000
_MAXLEN = 200
_DIM = 64
_BATCH = 4096

_NC, _NS = 2, 16
_NW = _NC * _NS
_BPW = _BATCH // _NW

_TBLK = 2048
_TGRID = (_VOCAB + _TBLK - 1) // _TBLK
_TROWS = _TGRID * (_TBLK // 128) * 128


def _tbody(in_ref, out_ref):
    eye = jnp.eye(_DIM, dtype=jnp.float32)
    t = lax.dot_general(in_ref[...], eye, (((0,), (0,)), ((), ())),
                        preferred_element_type=jnp.float32)
    t3 = t.reshape(_TBLK // 128, 128, _DIM)
    out_ref[:, :, 0:64] = t3[:, 0:64, :]
    out_ref[:, :, 64:128] = t3[:, 64:128, :]


def _tc_transpose(tt_cm):
    return pl.pallas_call(
        _tbody,
        out_shape=jax.ShapeDtypeStruct(
            (_TGRID * (_TBLK // 128), 64, 128), jnp.float32),
        grid=(_TGRID,),
        in_specs=[pl.BlockSpec((64, _TBLK), lambda g: (0, g))],
        out_specs=pl.BlockSpec((_TBLK // 128, 64, 128), lambda g: (g, 0, 0)),
    )(tt_cm)


def _body(xt_hbm, tok_hbm, pos_hbm, q_hbm,
          xv, pos_v, r0, r1, o0, o1, g0, g1, w0, w1):
    wid = lax.axis_index("s") * _NC + lax.axis_index("c")
    b0 = wid * _BPW
    rows = (r0, r1)
    outs = (o0, o1)
    gsems = (g0, g1)
    wsems = (w0, w1)

    pltpu.sync_copy(xt_hbm.at[:, pl.ds(b0, _BPW)], xv)
    pltpu.sync_copy(pos_hbm, pos_v)

    def remap_row(s2, carry):
        for kk in range(_BPW // 16):
            d = pl.ds(kk * 16, 16)
            v = xv[s2, d]
            xv[s2, d] = (((v >> 7) << 7) | ((v & 63) << 1) | ((v >> 6) & 1))
        return carry

    lax.fori_loop(0, _MAXLEN, remap_row, 0)

    def issue_gather(s, b):
        pltpu.async_copy(tok_hbm.at[xv.at[s]], rows[b], gsems[b])

    def wait_gather(s, b):
        pltpu.make_async_copy(tok_hbm.at[xv.at[s]], rows[b], gsems[b]).wait()

    def issue_write(s, b):
        pltpu.async_copy(outs[b], q_hbm.at[s, :, wid, :, :], wsems[b])

    def wait_write(s, b):
        pltpu.make_async_copy(outs[b], q_hbm.at[s, :, wid, :, :],
                              wsems[b]).wait()

    lane = lax.iota(jnp.int32, 16)
    issue_gather(0, 0)

    def s_body(s, carry):
        for b in range(2):
            sb = s * 2 + b
            pl.when(sb + 1 < _MAXLEN)(lambda: issue_gather(sb + 1, 1 - b))
            wait_gather(sb, b)
            pl.when(sb >= 2)(lambda: wait_write(sb - 2, b))

            pvs = [pos_v[sb, pl.ds(k * 16, 16)] for k in range(_DIM // 16)]
            cols = [16 * k + lane for k in range(_DIM // 16)]
            i0s = [(16 * k + lane) >> 3 for k in range(_DIM // 16)]
            i1 = lane & 7

            def j_body(j, c2):
                rowmix = (lane + j) & 15
                trows = [rowmix + 16 * g for g in range(_BPW // 16)]
                for ks in ((0, 1), (2, 3)):
                    vals = []
                    for g in range(_BPW // 16):
                        for k in ks:
                            vals.append(
                                (g, k,
                                 plsc.load_gather(rows[b],
                                                  [trows[g], cols[k]])
                                 + pvs[k]))
                    for g, k, v in vals:
                        plsc.store_scatter(outs[b], [i0s[k], i1, trows[g]], v)
                return c2

            lax.fori_loop(0, 16, j_body, 0)
            issue_write(sb, b)
        return carry

    lax.fori_loop(0, _MAXLEN // 2, s_body, 0)
    wait_write(_MAXLEN - 2, 0)
    wait_write(_MAXLEN - 1, 1)


@jax.jit
def _embed(xt, tt_sc, pos_table):
    mesh = plsc.VectorSubcoreMesh(core_axis_name="c", subcore_axis_name="s")
    run = pl.kernel(
        _body,
        out_type=jax.ShapeDtypeStruct(
            (_MAXLEN, _DIM // 8, _NW, 8, _BPW), jnp.float32),
        mesh=mesh,
        scratch_types=(
            [pltpu.VMEM((_MAXLEN, _BPW), jnp.int32),
             pltpu.VMEM((_MAXLEN, _DIM), jnp.float32)]
            + [pltpu.VMEM((_BPW, _DIM), jnp.float32)] * 2
            + [pltpu.VMEM((_DIM // 8, 8, _BPW), jnp.float32)] * 2
            + [pltpu.SemaphoreType.DMA] * 4
        ),
        compiler_params=pltpu.CompilerParams(
            use_tc_tiling_on_sc=False, needs_layout_passes=False),
    )
    return run(xt, tt_sc, pos_table)


def kernel(x, token_table, pos_table):
    xt = x.T.astype(jnp.int32)
    tt3 = _tc_transpose(token_table.T)
    tt_sc = tt3.reshape(_TROWS, _DIM)
    q = _embed(xt, tt_sc, pos_table)
    return q.transpose(2, 4, 0, 1, 3).reshape(_BATCH, _MAXLEN, _DIM)


# speedup vs baseline: 1.8772x; 1.3986x over previous
import jax
import jax.numpy as jnp
from jax import lax
from jax.experimental import pallas as pl
from jax.experimental.pallas import tpu as pltpu
from jax.experimental.pallas import tpu_sc as plsc

_VOCAB = 1000000
_MAXLEN = 200
_DIM = 64
_BATCH = 4096

_NC, _NS = 2, 16
_NW = _NC * _NS
_BPW = _BATCH // _NW

_TBLK = 2048
_TGRID = 245
_THALF = _TGRID * _TBLK
_TROWS = 2 * _THALF


def _tbody(a_ref, b_ref, out_ref):
    cat = jnp.concatenate([a_ref[...], b_ref[...]], axis=0)
    eye = jnp.eye(2 * _DIM, dtype=jnp.float32)
    out_ref[...] = lax.dot_general(cat, eye, (((0,), (0,)), ((), ())),
                                   preferred_element_type=jnp.float32)


def _tc_transpose(tt_cm):
    return pl.pallas_call(
        _tbody,
        out_shape=jax.ShapeDtypeStruct((_THALF, 2 * _DIM), jnp.float32),
        grid=(_TGRID,),
        in_specs=[pl.BlockSpec((_DIM, _TBLK), lambda g: (0, g)),
                  pl.BlockSpec((_DIM, _TBLK),
                               lambda g: (0, jnp.minimum(g + _TGRID,
                                                         _VOCAB // _TBLK)))],
        out_specs=pl.BlockSpec((_TBLK, 2 * _DIM), lambda g: (g, 0)),
    )(tt_cm, tt_cm)


def _body(xt_hbm, tok_hbm, pos_hbm, q_hbm,
          xv, pos_v, r0, r1, o0, o1, g0, g1, w0, w1):
    wid = lax.axis_index("s") * _NC + lax.axis_index("c")
    b0 = wid * _BPW
    rows = (r0, r1)
    outs = (o0, o1)
    gsems = (g0, g1)
    wsems = (w0, w1)

    pltpu.sync_copy(xt_hbm.at[:, pl.ds(b0, _BPW)], xv)
    pltpu.sync_copy(pos_hbm, pos_v)

    def remap_row(s2, carry):
        for kk in range(_BPW // 16):
            d = pl.ds(kk * 16, 16)
            v = xv[s2, d]
            h = (v >= _THALF).astype(jnp.int32)
            xv[s2, d] = ((v - _THALF * h) << 1) | h
        return carry

    lax.fori_loop(0, _MAXLEN, remap_row, 0)

    def issue_gather(s, b):
        pltpu.async_copy(tok_hbm.at[xv.at[s]], rows[b], gsems[b])

    def wait_gather(s, b):
        pltpu.make_async_copy(tok_hbm.at[xv.at[s]], rows[b], gsems[b]).wait()

    def issue_write(s, b):
        pltpu.async_copy(outs[b], q_hbm.at[s, :, wid, :, :], wsems[b])

    def wait_write(s, b):
        pltpu.make_async_copy(outs[b], q_hbm.at[s, :, wid, :, :],
                              wsems[b]).wait()

    lane = lax.iota(jnp.int32, 16)
    issue_gather(0, 0)

    def s_body(s, carry):
        for b in range(2):
            sb = s * 2 + b
            pl.when(sb + 1 < _MAXLEN)(lambda: issue_gather(sb + 1, 1 - b))
            wait_gather(sb, b)
            pl.when(sb >= 2)(lambda: wait_write(sb - 2, b))

            pvs = [pos_v[sb, pl.ds(k * 16, 16)] for k in range(_DIM // 16)]
            cols = [16 * k + lane for k in range(_DIM // 16)]
            i0s = [(16 * k + lane) >> 3 for k in range(_DIM // 16)]
            i1 = lane & 7

            def j_body(j, c2):
                rowmix = (lane + j) & 15
                trows = [rowmix + 16 * g for g in range(_BPW // 16)]
                for ks in ((0, 1), (2, 3)):
                    vals = []
                    for g in range(_BPW // 16):
                        for k in ks:
                            vals.append(
                                (g, k,
                                 plsc.load_gather(rows[b],
                                                  [trows[g], cols[k]])
                                 + pvs[k]))
                    for g, k, v in vals:
                        plsc.store_scatter(outs[b], [i0s[k], i1, trows[g]], v)
                return c2

            lax.fori_loop(0, 16, j_body, 0)
            issue_write(sb, b)
        return carry

    lax.fori_loop(0, _MAXLEN // 2, s_body, 0)
    wait_write(_MAXLEN - 2, 0)
    wait_write(_MAXLEN - 1, 1)


@jax.jit
def _embed(xt, tt_sc, pos_table):
    mesh = plsc.VectorSubcoreMesh(core_axis_name="c", subcore_axis_name="s")
    run = pl.kernel(
        _body,
        out_type=jax.ShapeDtypeStruct(
            (_MAXLEN, _DIM // 8, _NW, 8, _BPW), jnp.float32),
        mesh=mesh,
        scratch_types=(
            [pltpu.VMEM((_MAXLEN, _BPW), jnp.int32),
             pltpu.VMEM((_MAXLEN, _DIM), jnp.float32)]
            + [pltpu.VMEM((_BPW, _DIM), jnp.float32)] * 2
            + [pltpu.VMEM((_DIM // 8, 8, _BPW), jnp.float32)] * 2
            + [pltpu.SemaphoreType.DMA] * 4
        ),
        compiler_params=pltpu.CompilerParams(
            use_tc_tiling_on_sc=False, needs_layout_passes=False),
    )
    return run(xt, tt_sc, pos_table)


def kernel(x, token_table, pos_table):
    xt = x.T.astype(jnp.int32)
    tt3 = _tc_transpose(token_table.T)
    tt_sc = tt3.reshape(_TROWS, _DIM)
    q = _embed(xt, tt_sc, pos_table)
    return q.transpose(2, 4, 0, 1, 3).reshape(_BATCH, _MAXLEN, _DIM)
